# baseline (speedup 1.0000x reference)
import jax
import jax.numpy as jnp
from jax.experimental import pallas as pl

K = 1024


def _argmax_onehot_kernel(x_ref, o_ref):
    xb = x_ref[0]
    am = jnp.argmax(xb, axis=0)
    iota = jax.lax.broadcasted_iota(jnp.int32, xb.shape, 0)
    o_ref[0] = (iota == am[None, :]).astype(jnp.float32)


def kernel(x):
    B, Kdim, L = x.shape
    grid = (B,)
    return pl.pallas_call(
        _argmax_onehot_kernel,
        grid=grid,
        in_specs=[pl.BlockSpec((1, Kdim, L), lambda b: (b, 0, 0))],
        out_specs=pl.BlockSpec((1, Kdim, L), lambda b: (b, 0, 0)),
        out_shape=jax.ShapeDtypeStruct((B, Kdim, L), x.dtype),
    )(x)


# speedup vs baseline: 1.0255x; 1.0255x over previous
import jax
import jax.numpy as jnp
from jax.experimental import pallas as pl

K = 1024


BB = 4


def _argmax_onehot_kernel(x_ref, o_ref):
    xb = x_ref[...]
    am = jnp.argmax(xb, axis=1)
    iota = jax.lax.broadcasted_iota(jnp.int32, xb.shape, 1)
    o_ref[...] = (iota == am[:, None, :]).astype(jnp.float32)


def kernel(x):
    B, Kdim, L = x.shape
    grid = (B // BB,)
    return pl.pallas_call(
        _argmax_onehot_kernel,
        grid=grid,
        in_specs=[pl.BlockSpec((BB, Kdim, L), lambda b: (b, 0, 0))],
        out_specs=pl.BlockSpec((BB, Kdim, L), lambda b: (b, 0, 0)),
        out_shape=jax.ShapeDtypeStruct((B, Kdim, L), x.dtype),
    )(x)
